# baseline (speedup 1.0000x reference)
import functools

import jax
import jax.numpy as jnp
from jax import lax
from jax.experimental import pallas as pl
from jax.experimental.pallas import tpu as pltpu
from jax.experimental.pallas import tpu_sc as plsc

B, SEQ, D = 32, 1, 1024
E_ROUTED, E_SHARED, H, TOP_K = 64, 2, 512, 2
T = B * SEQ
NEG = -3.0e38


def _router_body(x_ref, rw_ref, logits_ref, logits_t_ref, gdense_ref):
    logits = jnp.dot(x_ref[...], rw_ref[...], preferred_element_type=jnp.float32)
    logits_ref[...] = logits
    logits_t_ref[...] = lax.dot_general(
        rw_ref[...], x_ref[...], (((0,), (1,)), ((), ())),
        preferred_element_type=jnp.float32)
    m = jnp.max(logits, axis=1, keepdims=True)
    e = jnp.exp(logits - m)
    p = e / jnp.sum(e, axis=1, keepdims=True)
    iota = jax.lax.broadcasted_iota(jnp.int32, (T, E_ROUTED), 1)
    m1 = jnp.max(p, axis=1, keepdims=True)
    i1 = jnp.min(jnp.where(p == m1, iota, E_ROUTED), axis=1, keepdims=True)
    sel1 = iota == i1
    p2 = jnp.where(sel1, -1.0, p)
    m2 = jnp.max(p2, axis=1, keepdims=True)
    i2 = jnp.min(jnp.where(p2 == m2, iota, E_ROUTED), axis=1, keepdims=True)
    sel2 = iota == i2
    gdense_ref[...] = jnp.where(sel1, m1, 0.0) + jnp.where(sel2, m2, 0.0)


def _routing_sc_body(lt_hbm, idx_hbm, gates_hbm, lt_v, idx_v, gates_v, sem):
    wid = lax.axis_index("s") * 2 + lax.axis_index("c")

    @pl.when(wid == 0)
    def _():
        pltpu.sync_copy(lt_hbm, lt_v)

        zf = jnp.zeros((16,), jnp.float32)
        zi = jnp.zeros((16,), jnp.int32)

        def top2_step(e, c):
            m1a, i1a, m2a, i2a, sa, m1b, i1b, m2b, i2b, sb = c
            ev = jnp.full((16,), e, jnp.int32)
            va = lt_v[e, pl.ds(0, 16)]
            vb = lt_v[e, pl.ds(16, 16)]
            g1 = va > m1a
            g2 = va > m2a
            mn = jnp.where(g1, va, m1a)
            sa = sa * jnp.exp(m1a - mn) + jnp.exp(va - mn)
            m2a = jnp.where(g1, m1a, jnp.where(g2, va, m2a))
            i2a = jnp.where(g1, i1a, jnp.where(g2, ev, i2a))
            m1a = mn
            i1a = jnp.where(g1, ev, i1a)
            g1 = vb > m1b
            g2 = vb > m2b
            mn = jnp.where(g1, vb, m1b)
            sb = sb * jnp.exp(m1b - mn) + jnp.exp(vb - mn)
            m2b = jnp.where(g1, m1b, jnp.where(g2, vb, m2b))
            i2b = jnp.where(g1, i1b, jnp.where(g2, ev, i2b))
            m1b = mn
            i1b = jnp.where(g1, ev, i1b)
            return (m1a, i1a, m2a, i2a, sa, m1b, i1b, m2b, i2b, sb)

        neg = jnp.full((16,), NEG, jnp.float32)
        m1a, i1a, m2a, i2a, sa, m1b, i1b, m2b, i2b, sb = lax.fori_loop(
            0, E_ROUTED, top2_step,
            (neg, zi, neg, zi, zf, neg, zi, neg, zi, zf), unroll=8)

        idx_v[0, pl.ds(0, 16)] = i1a
        idx_v[0, pl.ds(16, 16)] = i1b
        idx_v[1, pl.ds(0, 16)] = i2a
        idx_v[1, pl.ds(16, 16)] = i2b
        gates_v[0, pl.ds(0, 16)] = 1.0 / sa
        gates_v[0, pl.ds(16, 16)] = 1.0 / sb
        gates_v[1, pl.ds(0, 16)] = jnp.exp(m2a - m1a) / sa
        gates_v[1, pl.ds(16, 16)] = jnp.exp(m2b - m1b) / sb

        c1 = pltpu.async_copy(idx_v, idx_hbm, sem)
        c2 = pltpu.async_copy(gates_v, gates_hbm, sem)
        c1.wait()
        c2.wait()


_routing_sc = functools.partial(
    pl.kernel,
    out_type=(
        jax.ShapeDtypeStruct((TOP_K, T), jnp.int32),
        jax.ShapeDtypeStruct((TOP_K, T), jnp.float32),
    ),
    mesh=plsc.VectorSubcoreMesh(core_axis_name="c", subcore_axis_name="s"),
    compiler_params=pltpu.CompilerParams(needs_layout_passes=False),
    scratch_types=[
        pltpu.VMEM((E_ROUTED, T), jnp.float32),
        pltpu.VMEM((TOP_K, T), jnp.int32),
        pltpu.VMEM((TOP_K, T), jnp.float32),
        pltpu.SemaphoreType.DMA,
    ],
)(_routing_sc_body)


def _ffn(xb, wg_ref, wu_ref, wd_ref):
    hg = jnp.dot(xb, wg_ref.astype(jnp.bfloat16),
                 preferred_element_type=jnp.float32)
    hu = jnp.dot(xb, wu_ref.astype(jnp.bfloat16),
                 preferred_element_type=jnp.float32)
    h = hg * jax.lax.logistic(hg) * hu
    return jnp.dot(h.astype(jnp.bfloat16), wd_ref.astype(jnp.bfloat16),
                   preferred_element_type=jnp.float32)


def _moe_body(sched_ref, nact_ref, x_ref, g_ref,
              swg_ref, swu_ref, swd_ref, rwg_ref, rwu_ref, rwd_ref, out_ref):
    i = pl.program_id(0)
    xb = x_ref[...].astype(jnp.bfloat16)

    @pl.when(i == 0)
    def _shared():
        acc = jnp.zeros((T, D), jnp.float32)
        for e in range(E_SHARED):
            acc = acc + _ffn(xb, swg_ref[e], swu_ref[e], swd_ref[e])
        out_ref[...] = acc / float(E_SHARED)

    @pl.when(i < nact_ref[0])
    def _routed():
        eid = sched_ref[i]
        o = _ffn(xb, rwg_ref[0], rwu_ref[0], rwd_ref[0])
        iota = jax.lax.broadcasted_iota(jnp.int32, (T, E_ROUTED), 1)
        scale = jnp.sum(jnp.where(iota == eid, g_ref[...], 0.0),
                        axis=1, keepdims=True)
        out_ref[...] = out_ref[...] + o * scale


@jax.jit
def kernel(x, router_w, shared_wg, shared_wu, shared_wd,
           routed_wg, routed_wu, routed_wd):
    xf = x.reshape(T, D)

    logits, logits_t, g_dense = pl.pallas_call(
        _router_body,
        out_shape=(
            jax.ShapeDtypeStruct((T, E_ROUTED), jnp.float32),
            jax.ShapeDtypeStruct((E_ROUTED, T), jnp.float32),
            jax.ShapeDtypeStruct((T, E_ROUTED), jnp.float32),
        ),
    )(xf, router_w)

    idx_t, gates_t = _routing_sc(logits_t)
    indices = idx_t.T
    gates = gates_t.T

    ar = jnp.arange(E_ROUTED, dtype=jnp.int32)
    active = jnp.any(g_dense != 0.0, axis=0)
    order = jnp.argsort(jnp.where(active, ar, ar + E_ROUTED)).astype(jnp.int32)
    nact = jnp.sum(active).astype(jnp.int32)
    last = order[nact - 1]
    sched = jnp.where(ar < nact, order, last)

    out = pl.pallas_call(
        _moe_body,
        grid_spec=pltpu.PrefetchScalarGridSpec(
            num_scalar_prefetch=2,
            grid=(E_ROUTED,),
            in_specs=[
                pl.BlockSpec((T, D), lambda i, sched, nact: (0, 0)),
                pl.BlockSpec((T, E_ROUTED), lambda i, sched, nact: (0, 0)),
                pl.BlockSpec((E_SHARED, D, H), lambda i, sched, nact: (0, 0, 0)),
                pl.BlockSpec((E_SHARED, D, H), lambda i, sched, nact: (0, 0, 0)),
                pl.BlockSpec((E_SHARED, H, D), lambda i, sched, nact: (0, 0, 0)),
                pl.BlockSpec((1, D, H), lambda i, sched, nact: (sched[i], 0, 0)),
                pl.BlockSpec((1, D, H), lambda i, sched, nact: (sched[i], 0, 0)),
                pl.BlockSpec((1, H, D), lambda i, sched, nact: (sched[i], 0, 0)),
            ],
            out_specs=pl.BlockSpec((T, D), lambda i, sched, nact: (0, 0)),
        ),
        out_shape=jax.ShapeDtypeStruct((T, D), jnp.float32),
        compiler_params=pltpu.CompilerParams(
            dimension_semantics=("arbitrary",),
        ),
    )(sched, nact.reshape(1), xf, g_dense,
      shared_wg, shared_wu, shared_wd, routed_wg, routed_wu, routed_wd)

    return out.reshape(B, SEQ, D), logits, indices, gates


# speedup vs baseline: 1.0869x; 1.0869x over previous
import functools

import jax
import jax.numpy as jnp
from jax import lax
from jax.experimental import pallas as pl
from jax.experimental.pallas import tpu as pltpu
from jax.experimental.pallas import tpu_sc as plsc

B, SEQ, D = 32, 1, 1024
E_ROUTED, E_SHARED, H, TOP_K = 64, 2, 512, 2
T = B * SEQ
NEG = -3.0e38


def _router_body(x_ref, rw_ref, logits_ref, logits_t_ref, gdense_ref,
                 sched_ref, nact_ref):
    logits = jnp.dot(x_ref[...], rw_ref[...], preferred_element_type=jnp.float32)
    logits_ref[...] = logits
    logits_t_ref[...] = lax.dot_general(
        rw_ref[...], x_ref[...], (((0,), (1,)), ((), ())),
        preferred_element_type=jnp.float32)
    m = jnp.max(logits, axis=1, keepdims=True)
    e = jnp.exp(logits - m)
    p = e / jnp.sum(e, axis=1, keepdims=True)
    iota = jax.lax.broadcasted_iota(jnp.int32, (T, E_ROUTED), 1)
    m1 = jnp.max(p, axis=1, keepdims=True)
    i1 = jnp.min(jnp.where(p == m1, iota, E_ROUTED), axis=1, keepdims=True)
    sel1 = iota == i1
    p2 = jnp.where(sel1, -1.0, p)
    m2 = jnp.max(p2, axis=1, keepdims=True)
    i2 = jnp.min(jnp.where(p2 == m2, iota, E_ROUTED), axis=1, keepdims=True)
    sel2 = iota == i2
    gdense_ref[...] = jnp.where(sel1, m1, 0.0) + jnp.where(sel2, m2, 0.0)

    af = jnp.max(jnp.where(sel1 | sel2, 1.0, 0.0), axis=0, keepdims=True)
    rr = jax.lax.broadcasted_iota(jnp.int32, (E_ROUTED, E_ROUTED), 0)
    cc = jax.lax.broadcasted_iota(jnp.int32, (E_ROUTED, E_ROUTED), 1)
    lt = jnp.where(rr <= cc, 1.0, 0.0)
    incl = jnp.dot(af, lt, preferred_element_type=jnp.float32)
    pos = incl - af
    nact = incl[:, E_ROUTED - 1:E_ROUTED]
    er = jax.lax.broadcasted_iota(jnp.int32, (1, E_ROUTED), 1).astype(jnp.float32)
    hit_pe = jnp.where((rr.astype(jnp.float32) == pos) & (af > 0.0), 1.0, 0.0)
    sched_f = jnp.dot(hit_pe * er, jnp.ones((E_ROUTED, 1), jnp.float32),
                      preferred_element_type=jnp.float32)
    last = jnp.max(jnp.where(af > 0.0, er, -1.0), axis=1, keepdims=True)
    slot = jax.lax.broadcasted_iota(jnp.int32, (E_ROUTED, 1), 0).astype(jnp.float32)
    sched_ref[...] = jnp.where(slot < nact, sched_f, last).astype(jnp.int32)
    nact_ref[...] = nact.astype(jnp.int32)


def _routing_sc_body(lt_hbm, idx_hbm, gates_hbm, lt_v, idx_v, gates_v, sem):
    wid = lax.axis_index("s") * 2 + lax.axis_index("c")

    @pl.when(wid == 0)
    def _():
        pltpu.sync_copy(lt_hbm, lt_v)

        zf = jnp.zeros((16,), jnp.float32)
        zi = jnp.zeros((16,), jnp.int32)

        def top2_step(e, c):
            m1a, i1a, m2a, i2a, sa, m1b, i1b, m2b, i2b, sb = c
            ev = jnp.full((16,), e, jnp.int32)
            va = lt_v[e, pl.ds(0, 16)]
            vb = lt_v[e, pl.ds(16, 16)]
            g1 = va > m1a
            g2 = va > m2a
            mn = jnp.where(g1, va, m1a)
            sa = sa * jnp.exp(m1a - mn) + jnp.exp(va - mn)
            m2a = jnp.where(g1, m1a, jnp.where(g2, va, m2a))
            i2a = jnp.where(g1, i1a, jnp.where(g2, ev, i2a))
            m1a = mn
            i1a = jnp.where(g1, ev, i1a)
            g1 = vb > m1b
            g2 = vb > m2b
            mn = jnp.where(g1, vb, m1b)
            sb = sb * jnp.exp(m1b - mn) + jnp.exp(vb - mn)
            m2b = jnp.where(g1, m1b, jnp.where(g2, vb, m2b))
            i2b = jnp.where(g1, i1b, jnp.where(g2, ev, i2b))
            m1b = mn
            i1b = jnp.where(g1, ev, i1b)
            return (m1a, i1a, m2a, i2a, sa, m1b, i1b, m2b, i2b, sb)

        neg = jnp.full((16,), NEG, jnp.float32)
        m1a, i1a, m2a, i2a, sa, m1b, i1b, m2b, i2b, sb = lax.fori_loop(
            0, E_ROUTED, top2_step,
            (neg, zi, neg, zi, zf, neg, zi, neg, zi, zf), unroll=8)

        idx_v[0, pl.ds(0, 16)] = i1a
        idx_v[0, pl.ds(16, 16)] = i1b
        idx_v[1, pl.ds(0, 16)] = i2a
        idx_v[1, pl.ds(16, 16)] = i2b
        gates_v[0, pl.ds(0, 16)] = 1.0 / sa
        gates_v[0, pl.ds(16, 16)] = 1.0 / sb
        gates_v[1, pl.ds(0, 16)] = jnp.exp(m2a - m1a) / sa
        gates_v[1, pl.ds(16, 16)] = jnp.exp(m2b - m1b) / sb

        c1 = pltpu.async_copy(idx_v, idx_hbm, sem)
        c2 = pltpu.async_copy(gates_v, gates_hbm, sem)
        c1.wait()
        c2.wait()


@functools.cache
def _routing_sc():
    return functools.partial(
        pl.kernel,
        out_type=(
            jax.ShapeDtypeStruct((TOP_K, T), jnp.int32),
            jax.ShapeDtypeStruct((TOP_K, T), jnp.float32),
        ),
        mesh=plsc.VectorSubcoreMesh(core_axis_name="c", subcore_axis_name="s"),
        compiler_params=pltpu.CompilerParams(needs_layout_passes=False),
        scratch_types=[
            pltpu.VMEM((E_ROUTED, T), jnp.float32),
            pltpu.VMEM((TOP_K, T), jnp.int32),
            pltpu.VMEM((TOP_K, T), jnp.float32),
            pltpu.SemaphoreType.DMA,
        ],
    )(_routing_sc_body)


def _ffn(xb, wg_ref, wu_ref, wd_ref):
    hg = jnp.dot(xb, wg_ref.astype(jnp.bfloat16),
                 preferred_element_type=jnp.float32)
    hu = jnp.dot(xb, wu_ref.astype(jnp.bfloat16),
                 preferred_element_type=jnp.float32)
    h = hg * jax.lax.logistic(hg) * hu
    return jnp.dot(h.astype(jnp.bfloat16), wd_ref.astype(jnp.bfloat16),
                   preferred_element_type=jnp.float32)


def _moe_body(sched_ref, nact_ref, x_ref, g_ref,
              swg_ref, swu_ref, swd_ref, rwg_ref, rwu_ref, rwd_ref, out_ref):
    i = pl.program_id(0)
    xb = x_ref[...].astype(jnp.bfloat16)

    @pl.when(i == 0)
    def _shared():
        acc = jnp.zeros((T, D), jnp.float32)
        for e in range(E_SHARED):
            acc = acc + _ffn(xb, swg_ref[e], swu_ref[e], swd_ref[e])
        out_ref[...] = acc / float(E_SHARED)

    @pl.when(i < nact_ref[0])
    def _routed():
        eid = sched_ref[i]
        o = _ffn(xb, rwg_ref[0], rwu_ref[0], rwd_ref[0])
        iota = jax.lax.broadcasted_iota(jnp.int32, (T, E_ROUTED), 1)
        scale = jnp.sum(jnp.where(iota == eid, g_ref[...], 0.0),
                        axis=1, keepdims=True)
        out_ref[...] = out_ref[...] + o * scale


@jax.jit
def kernel(x, router_w, shared_wg, shared_wu, shared_wd,
           routed_wg, routed_wu, routed_wd):
    xf = x.reshape(T, D)

    logits, logits_t, g_dense, sched_col, nact_col = pl.pallas_call(
        _router_body,
        out_shape=(
            jax.ShapeDtypeStruct((T, E_ROUTED), jnp.float32),
            jax.ShapeDtypeStruct((E_ROUTED, T), jnp.float32),
            jax.ShapeDtypeStruct((T, E_ROUTED), jnp.float32),
            jax.ShapeDtypeStruct((E_ROUTED, 1), jnp.int32),
            jax.ShapeDtypeStruct((1, 1), jnp.int32),
        ),
    )(xf, router_w)

    idx_t, gates_t = _routing_sc()(logits_t)
    indices = idx_t.T
    gates = gates_t.T

    sched = sched_col.reshape(E_ROUTED)
    nact = nact_col.reshape(1)

    out = pl.pallas_call(
        _moe_body,
        grid_spec=pltpu.PrefetchScalarGridSpec(
            num_scalar_prefetch=2,
            grid=(E_ROUTED,),
            in_specs=[
                pl.BlockSpec((T, D), lambda i, sched, nact: (0, 0)),
                pl.BlockSpec((T, E_ROUTED), lambda i, sched, nact: (0, 0)),
                pl.BlockSpec((E_SHARED, D, H), lambda i, sched, nact: (0, 0, 0)),
                pl.BlockSpec((E_SHARED, D, H), lambda i, sched, nact: (0, 0, 0)),
                pl.BlockSpec((E_SHARED, H, D), lambda i, sched, nact: (0, 0, 0)),
                pl.BlockSpec((1, D, H), lambda i, sched, nact: (sched[i], 0, 0)),
                pl.BlockSpec((1, D, H), lambda i, sched, nact: (sched[i], 0, 0)),
                pl.BlockSpec((1, H, D), lambda i, sched, nact: (sched[i], 0, 0)),
            ],
            out_specs=pl.BlockSpec((T, D), lambda i, sched, nact: (0, 0)),
        ),
        out_shape=jax.ShapeDtypeStruct((T, D), jnp.float32),
        compiler_params=pltpu.CompilerParams(
            dimension_semantics=("arbitrary",),
        ),
    )(sched, nact, xf, g_dense,
      shared_wg, shared_wu, shared_wd, routed_wg, routed_wu, routed_wd)

    return out.reshape(B, SEQ, D), logits, indices, gates
